# baseline (speedup 1.0000x reference)
import functools

import jax
import jax.numpy as jnp
from jax.experimental import pallas as pl
from jax.experimental.pallas import tpu as pltpu

_N = 10000
_BLOCK = 1000


def _fused_kernel(x_ref, wz_ref, wh_ref, bz_ref, bh_ref, wlin_ref, blin_ref,
                  out_ref):
    x = x_ref[...]
    z_logit = jnp.dot(x, wz_ref[...], preferred_element_type=jnp.float32)
    h_logit = jnp.dot(x, wh_ref[...], preferred_element_type=jnp.float32)
    z = jax.nn.sigmoid(z_logit + bz_ref[...])
    t = jnp.tanh(h_logit + bh_ref[...])
    h = jax.nn.relu((1.0 - z) * t)
    out = jnp.sum(h * wlin_ref[...], axis=1, keepdims=True) + blin_ref[...]
    out_ref[...] = out


@functools.partial(jax.jit, static_argnames=())
def kernel(x, edge_index, edge_weight, W_xz, b_xz, W_hz, b_hz, W_xr, b_xr,
           W_hr, b_hr, W_xh, b_xh, W_hh, b_hh, W_lin, b_lin):
    n, f_in = x.shape
    hid = W_xz.shape[1]
    bz = (b_xz + b_hz).reshape(1, hid)
    bh = (b_xh + b_hh).reshape(1, hid)
    wlin = W_lin.reshape(1, hid)
    blin = b_lin.reshape(1, 1)

    grid = n // _BLOCK
    rep = lambda i: (0, 0)
    out = pl.pallas_call(
        _fused_kernel,
        grid=(grid,),
        in_specs=[
            pl.BlockSpec((_BLOCK, f_in), lambda i: (i, 0)),
            pl.BlockSpec((f_in, hid), rep),
            pl.BlockSpec((f_in, hid), rep),
            pl.BlockSpec((1, hid), rep),
            pl.BlockSpec((1, hid), rep),
            pl.BlockSpec((1, hid), rep),
            pl.BlockSpec((1, 1), rep),
        ],
        out_specs=pl.BlockSpec((_BLOCK, 1), lambda i: (i, 0)),
        out_shape=jax.ShapeDtypeStruct((n, 1), x.dtype),
        compiler_params=pltpu.CompilerParams(
            dimension_semantics=("arbitrary",),
        ),
    )(x, W_xz, W_xh, bz, bh, wlin, blin)
    return out


# speedup vs baseline: 1.0095x; 1.0095x over previous
import functools

import jax
import jax.numpy as jnp
from jax.experimental import pallas as pl
from jax.experimental.pallas import tpu as pltpu

_N = 10000
_BLOCK = 1000


def _fused_kernel(x_ref, wz_ref, wh_ref, bz_ref, bh_ref, wlin_ref, blin_ref,
                  out_ref):
    x = x_ref[...]
    z_logit = jnp.dot(x, wz_ref[...], preferred_element_type=jnp.float32)
    h_logit = jnp.dot(x, wh_ref[...], preferred_element_type=jnp.float32)
    z = jax.nn.sigmoid(z_logit + bz_ref[...])
    t = jnp.tanh(h_logit + bh_ref[...])
    h = jax.nn.relu((1.0 - z) * t)
    out = jnp.sum(h * wlin_ref[...], axis=1, keepdims=True) + blin_ref[...]
    out_ref[...] = out


@functools.partial(jax.jit, static_argnames=())
def kernel(x, edge_index, edge_weight, W_xz, b_xz, W_hz, b_hz, W_xr, b_xr,
           W_hr, b_hr, W_xh, b_xh, W_hh, b_hh, W_lin, b_lin):
    n, f_in = x.shape
    hid = W_xz.shape[1]
    bz = (b_xz + b_hz).reshape(1, hid)
    bh = (b_xh + b_hh).reshape(1, hid)
    wlin = W_lin.reshape(1, hid)
    blin = b_lin.reshape(1, 1)

    grid = n // _BLOCK
    rep = lambda i: (0, 0)
    out = pl.pallas_call(
        _fused_kernel,
        grid=(grid,),
        in_specs=[
            pl.BlockSpec((_BLOCK, f_in), lambda i: (i, 0)),
            pl.BlockSpec((f_in, hid), rep),
            pl.BlockSpec((f_in, hid), rep),
            pl.BlockSpec((1, hid), rep),
            pl.BlockSpec((1, hid), rep),
            pl.BlockSpec((1, hid), rep),
            pl.BlockSpec((1, 1), rep),
        ],
        out_specs=pl.BlockSpec((_BLOCK, 1), lambda i: (i, 0)),
        out_shape=jax.ShapeDtypeStruct((n, 1), x.dtype),
        compiler_params=pltpu.CompilerParams(
            dimension_semantics=("parallel",),
        ),
    )(x, W_xz, W_xh, bz, bh, wlin, blin)
    return out


# speedup vs baseline: 1.1806x; 1.1694x over previous
import functools

import jax
import jax.numpy as jnp
from jax.experimental import pallas as pl
from jax.experimental.pallas import tpu as pltpu

_N = 10000
_BLOCK = 2000


def _fused_kernel(x_ref, wz_ref, wh_ref, bz_ref, bh_ref, wlin_ref, blin_ref,
                  out_ref):
    x = x_ref[...]
    z_logit = jnp.dot(x, wz_ref[...], preferred_element_type=jnp.float32)
    h_logit = jnp.dot(x, wh_ref[...], preferred_element_type=jnp.float32)
    z = jax.nn.sigmoid(z_logit + bz_ref[...])
    t = jnp.tanh(h_logit + bh_ref[...])
    h = jax.nn.relu((1.0 - z) * t)
    out = jnp.sum(h * wlin_ref[...], axis=1, keepdims=True) + blin_ref[...]
    out_ref[...] = out


@functools.partial(jax.jit, static_argnames=())
def kernel(x, edge_index, edge_weight, W_xz, b_xz, W_hz, b_hz, W_xr, b_xr,
           W_hr, b_hr, W_xh, b_xh, W_hh, b_hh, W_lin, b_lin):
    n, f_in = x.shape
    hid = W_xz.shape[1]
    bz = (b_xz + b_hz).reshape(1, hid)
    bh = (b_xh + b_hh).reshape(1, hid)
    wlin = W_lin.reshape(1, hid)
    blin = b_lin.reshape(1, 1)

    grid = n // _BLOCK
    rep = lambda i: (0, 0)
    out = pl.pallas_call(
        _fused_kernel,
        grid=(grid,),
        in_specs=[
            pl.BlockSpec((_BLOCK, f_in), lambda i: (i, 0)),
            pl.BlockSpec((f_in, hid), rep),
            pl.BlockSpec((f_in, hid), rep),
            pl.BlockSpec((1, hid), rep),
            pl.BlockSpec((1, hid), rep),
            pl.BlockSpec((1, hid), rep),
            pl.BlockSpec((1, 1), rep),
        ],
        out_specs=pl.BlockSpec((_BLOCK, 1), lambda i: (i, 0)),
        out_shape=jax.ShapeDtypeStruct((n, 1), x.dtype),
        compiler_params=pltpu.CompilerParams(
            dimension_semantics=("parallel",),
        ),
    )(x, W_xz, W_xh, bz, bh, wlin, blin)
    return out


# speedup vs baseline: 1.2674x; 1.0735x over previous
import functools

import jax
import jax.numpy as jnp
from jax.experimental import pallas as pl
from jax.experimental.pallas import tpu as pltpu

_N = 10000
_BLOCK = 5000


def _fused_kernel(x_ref, wz_ref, wh_ref, bz_ref, bh_ref, wlin_ref, blin_ref,
                  out_ref):
    x = x_ref[...]
    z_logit = jnp.dot(x, wz_ref[...], preferred_element_type=jnp.float32)
    h_logit = jnp.dot(x, wh_ref[...], preferred_element_type=jnp.float32)
    z = jax.nn.sigmoid(z_logit + bz_ref[...])
    t = jnp.tanh(h_logit + bh_ref[...])
    h = jax.nn.relu((1.0 - z) * t)
    out = jnp.sum(h * wlin_ref[...], axis=1, keepdims=True) + blin_ref[...]
    out_ref[...] = out


@functools.partial(jax.jit, static_argnames=())
def kernel(x, edge_index, edge_weight, W_xz, b_xz, W_hz, b_hz, W_xr, b_xr,
           W_hr, b_hr, W_xh, b_xh, W_hh, b_hh, W_lin, b_lin):
    n, f_in = x.shape
    hid = W_xz.shape[1]
    bz = (b_xz + b_hz).reshape(1, hid)
    bh = (b_xh + b_hh).reshape(1, hid)
    wlin = W_lin.reshape(1, hid)
    blin = b_lin.reshape(1, 1)

    grid = n // _BLOCK
    rep = lambda i: (0, 0)
    out = pl.pallas_call(
        _fused_kernel,
        grid=(grid,),
        in_specs=[
            pl.BlockSpec((_BLOCK, f_in), lambda i: (i, 0)),
            pl.BlockSpec((f_in, hid), rep),
            pl.BlockSpec((f_in, hid), rep),
            pl.BlockSpec((1, hid), rep),
            pl.BlockSpec((1, hid), rep),
            pl.BlockSpec((1, hid), rep),
            pl.BlockSpec((1, 1), rep),
        ],
        out_specs=pl.BlockSpec((_BLOCK, 1), lambda i: (i, 0)),
        out_shape=jax.ShapeDtypeStruct((n, 1), x.dtype),
        compiler_params=pltpu.CompilerParams(
            dimension_semantics=("parallel",),
        ),
    )(x, W_xz, W_xh, bz, bh, wlin, blin)
    return out
